# baseline (speedup 1.0000x reference)
import functools

import jax
import jax.numpy as jnp
from jax import lax
from jax.experimental import pallas as pl
from jax.experimental.pallas import tpu as pltpu
from jax.experimental.pallas import tpu_sc as plsc

BATCH = 4096
EMBED = 32
LANE = 128
NBUF = 4
BTAIL = 8

_info = plsc.get_sparse_core_info()
_NC = _info.num_cores
_NS = _info.num_subcores
_NW = _NC * _NS
_BPW = BATCH // _NW


def _gather_body(pt_hbm, qt_hbm, but_hbm, bit_hbm, uid_hbm, iid_hbm,
                 pta_out, qta_out, tail_out,
                 uidx_v, iidx_v, zbuf,
                 pslabs, qslabs, bslabs, cslabs,
                 shp, shq, shb,
                 psems, qsems, bsems, csems, lsems, zsem):
    tid = lax.axis_index("s")
    wid = tid * _NC + lax.axis_index("c")
    base = pl.multiple_of(wid * _BPW, _BPW)

    pltpu.sync_copy(uid_hbm.at[pl.ds(base, _BPW)], uidx_v)
    pltpu.sync_copy(iid_hbm.at[pl.ds(base, _BPW)], iidx_v)

    zeros = jnp.zeros((16,), jnp.float32)
    for r in range(BTAIL - 2):
        for j in range(LANE // 16):
            zbuf[r, pl.ds(j * 16, 16)] = zeros
    cpz = pltpu.async_copy(zbuf, shb.at[tid, pl.ds(2, BTAIL - 2), :], zsem)

    def fire(u, v, slot):
        uoff = pl.multiple_of((u >> 7) * LANE, LANE)
        voff = pl.multiple_of((v >> 7) * LANE, LANE)
        pltpu.async_copy(pt_hbm.at[:, pl.ds(uoff, LANE)], pslabs[slot],
                         psems[slot])
        pltpu.async_copy(qt_hbm.at[:, pl.ds(voff, LANE)], qslabs[slot],
                         qsems[slot])
        pltpu.async_copy(but_hbm.at[:, pl.ds(uoff, LANE)], bslabs[slot],
                         bsems[slot])
        pltpu.async_copy(bit_hbm.at[:, pl.ds(voff, LANE)], cslabs[slot],
                         csems[slot])

    def wait_slabs(slot):
        pltpu.make_async_copy(pt_hbm.at[:, pl.ds(0, LANE)],
                              pslabs[slot], psems[slot]).wait()
        pltpu.make_async_copy(qt_hbm.at[:, pl.ds(0, LANE)],
                              qslabs[slot], qsems[slot]).wait()
        pltpu.make_async_copy(but_hbm.at[:, pl.ds(0, LANE)],
                              bslabs[slot], bsems[slot]).wait()
        pltpu.make_async_copy(bit_hbm.at[:, pl.ds(0, LANE)],
                              cslabs[slot], csems[slot]).wait()

    def extract(u, v, i, slot):
        ul = u & (LANE - 1)
        vl = v & (LANE - 1)
        pltpu.async_copy(pslabs[slot].at[:, pl.ds(ul, 1)],
                         shp.at[tid, :, pl.ds(i, 1)], lsems[slot])
        pltpu.async_copy(qslabs[slot].at[:, pl.ds(vl, 1)],
                         shq.at[tid, :, pl.ds(i, 1)], lsems[slot])
        pltpu.async_copy(bslabs[slot].at[:, pl.ds(ul, 1)],
                         shb.at[tid, pl.ds(0, 1), pl.ds(i, 1)], lsems[slot])
        pltpu.async_copy(cslabs[slot].at[:, pl.ds(vl, 1)],
                         shb.at[tid, pl.ds(1, 1), pl.ds(i, 1)], lsems[slot])

    def drain_extract(slot):
        pltpu.make_async_copy(
            pslabs[slot].at[:, pl.ds(0, 1)],
            shp.at[tid, :, pl.ds(0, 1)], lsems[slot]).wait()
        pltpu.make_async_copy(
            qslabs[slot].at[:, pl.ds(0, 1)],
            shq.at[tid, :, pl.ds(0, 1)], lsems[slot]).wait()
        pltpu.make_async_copy(
            bslabs[slot].at[:, pl.ds(0, 1)],
            shb.at[tid, pl.ds(0, 1), pl.ds(0, 1)], lsems[slot]).wait()
        pltpu.make_async_copy(
            cslabs[slot].at[:, pl.ds(0, 1)],
            shb.at[tid, pl.ds(1, 1), pl.ds(0, 1)], lsems[slot]).wait()

    def group(g, carry):
        goff = pl.multiple_of(g * 16, 16)
        uv = uidx_v[pl.ds(goff, 16)]
        vv = iidx_v[pl.ds(goff, 16)]
        for j in range(NBUF):
            fire(uv[j], vv[j], j)
        for j in range(16):
            slot = j % NBUF
            wait_slabs(slot)
            extract(uv[j], vv[j], goff + j, slot)
            drain_extract(slot)
            if j + NBUF < 16:
                fire(uv[j + NBUF], vv[j + NBUF], slot)
        return carry

    lax.fori_loop(0, _BPW // 16, group, 0)

    pltpu.make_async_copy(zbuf, shb.at[tid, pl.ds(2, BTAIL - 2), :],
                          zsem).wait()
    pltpu.sync_copy(shp.at[tid], pta_out.at[:, pl.ds(base, _BPW)])
    pltpu.sync_copy(shq.at[tid], qta_out.at[:, pl.ds(base, _BPW)])
    pltpu.sync_copy(shb.at[tid], tail_out.at[:, pl.ds(base, _BPW)])


_gather = pl.kernel(
    _gather_body,
    out_type=(
        jax.ShapeDtypeStruct((EMBED, BATCH), jnp.float32),
        jax.ShapeDtypeStruct((EMBED, BATCH), jnp.float32),
        jax.ShapeDtypeStruct((BTAIL, BATCH), jnp.float32),
    ),
    mesh=plsc.VectorSubcoreMesh(core_axis_name="c", subcore_axis_name="s"),
    scratch_types=[
        pltpu.VMEM((_BPW,), jnp.int32),
        pltpu.VMEM((_BPW,), jnp.int32),
        pltpu.VMEM((BTAIL - 2, LANE), jnp.float32),
        [pltpu.VMEM((EMBED, LANE), jnp.float32) for _ in range(NBUF)],
        [pltpu.VMEM((EMBED, LANE), jnp.float32) for _ in range(NBUF)],
        [pltpu.VMEM((1, LANE), jnp.float32) for _ in range(NBUF)],
        [pltpu.VMEM((1, LANE), jnp.float32) for _ in range(NBUF)],
        pltpu.VMEM_SHARED((_NS, EMBED, LANE), jnp.float32),
        pltpu.VMEM_SHARED((_NS, EMBED, LANE), jnp.float32),
        pltpu.VMEM_SHARED((_NS, BTAIL, LANE), jnp.float32),
        [pltpu.SemaphoreType.DMA for _ in range(NBUF)],
        [pltpu.SemaphoreType.DMA for _ in range(NBUF)],
        [pltpu.SemaphoreType.DMA for _ in range(NBUF)],
        [pltpu.SemaphoreType.DMA for _ in range(NBUF)],
        [pltpu.SemaphoreType.DMA for _ in range(NBUF)],
        pltpu.SemaphoreType.DMA,
    ],
    compiler_params=pltpu.CompilerParams(use_tc_tiling_on_sc=True),
)


_BM = 1024


def _score_body(p_ref, q_ref, t_ref, o_ref):
    acc = lax.dot_general(
        p_ref[...], q_ref[...],
        (((0,), (0,)), ((), ())),
        preferred_element_type=jnp.float32,
    )
    ones = jnp.ones((BTAIL, BATCH), jnp.float32)
    o_ref[...] = acc + lax.dot_general(
        t_ref[...], ones,
        (((0,), (0,)), ((), ())),
        preferred_element_type=jnp.float32,
    )


@jax.jit
def _score(pta, qta, tail):
    return pl.pallas_call(
        _score_body,
        grid=(BATCH // _BM,),
        in_specs=[
            pl.BlockSpec((EMBED, _BM), lambda i: (0, i)),
            pl.BlockSpec((EMBED, BATCH), lambda i: (0, 0)),
            pl.BlockSpec((BTAIL, _BM), lambda i: (0, i)),
        ],
        out_specs=pl.BlockSpec((_BM, BATCH), lambda i: (i, 0)),
        out_shape=jax.ShapeDtypeStruct((BATCH, BATCH), jnp.float32),
        compiler_params=pltpu.CompilerParams(
            dimension_semantics=("arbitrary",),
        ),
    )(pta, qta, tail)


@jax.jit
def kernel(user_ids, item_ids, P, Q, B_user, B_item):
    uid = user_ids.astype(jnp.int32)
    iid = item_ids.astype(jnp.int32)
    pta, qta, tail = _gather(P.T, Q.T, B_user.T, B_item.T, uid, iid)
    return _score(pta, qta, tail)


# speedup vs baseline: 1.0165x; 1.0165x over previous
import functools

import jax
import jax.numpy as jnp
from jax import lax
from jax.experimental import pallas as pl
from jax.experimental.pallas import tpu as pltpu
from jax.experimental.pallas import tpu_sc as plsc

BATCH = 4096
EMBED = 32
LANE = 128
NBUF = 4
BTAIL = 8

_info = plsc.get_sparse_core_info()
_NC = _info.num_cores
_NS = _info.num_subcores
_NW = _NC * _NS
_BPW = BATCH // _NW


def _gather_body(pt_hbm, qt_hbm, but_hbm, bit_hbm, uid_hbm, iid_hbm,
                 pta_out, qta_out, tail_out,
                 uidx_v, iidx_v, zbuf,
                 pslabs, qslabs, bslabs, cslabs,
                 shp, shq, shb,
                 psems, qsems, bsems, csems, lsems, zsem):
    tid = lax.axis_index("s")
    wid = tid * _NC + lax.axis_index("c")
    base = pl.multiple_of(wid * _BPW, _BPW)

    pltpu.sync_copy(uid_hbm.at[pl.ds(base, _BPW)], uidx_v)
    pltpu.sync_copy(iid_hbm.at[pl.ds(base, _BPW)], iidx_v)

    zeros = jnp.zeros((16,), jnp.float32)
    for r in range(BTAIL - 2):
        for j in range(LANE // 16):
            zbuf[r, pl.ds(j * 16, 16)] = zeros
    cpz = pltpu.async_copy(zbuf, shb.at[tid, pl.ds(2, BTAIL - 2), :], zsem)

    def fire(u, v, slot):
        uoff = pl.multiple_of((u >> 7) * LANE, LANE)
        voff = pl.multiple_of((v >> 7) * LANE, LANE)
        pltpu.async_copy(pt_hbm.at[:, pl.ds(uoff, LANE)], pslabs[slot],
                         psems[slot])
        pltpu.async_copy(qt_hbm.at[:, pl.ds(voff, LANE)], qslabs[slot],
                         qsems[slot])
        pltpu.async_copy(but_hbm.at[:, pl.ds(uoff, LANE)], bslabs[slot],
                         bsems[slot])
        pltpu.async_copy(bit_hbm.at[:, pl.ds(voff, LANE)], cslabs[slot],
                         csems[slot])

    def wait_slabs(slot):
        pltpu.make_async_copy(pt_hbm.at[:, pl.ds(0, LANE)],
                              pslabs[slot], psems[slot]).wait()
        pltpu.make_async_copy(qt_hbm.at[:, pl.ds(0, LANE)],
                              qslabs[slot], qsems[slot]).wait()
        pltpu.make_async_copy(but_hbm.at[:, pl.ds(0, LANE)],
                              bslabs[slot], bsems[slot]).wait()
        pltpu.make_async_copy(bit_hbm.at[:, pl.ds(0, LANE)],
                              cslabs[slot], csems[slot]).wait()

    def extract(u, v, i, slot):
        ul = u & (LANE - 1)
        vl = v & (LANE - 1)
        pltpu.async_copy(pslabs[slot].at[:, pl.ds(ul, 1)],
                         shp.at[tid, :, pl.ds(i, 1)], lsems[slot])
        pltpu.async_copy(qslabs[slot].at[:, pl.ds(vl, 1)],
                         shq.at[tid, :, pl.ds(i, 1)], lsems[slot])
        pltpu.async_copy(bslabs[slot].at[:, pl.ds(ul, 1)],
                         shb.at[tid, pl.ds(0, 1), pl.ds(i, 1)], lsems[slot])
        pltpu.async_copy(cslabs[slot].at[:, pl.ds(vl, 1)],
                         shb.at[tid, pl.ds(1, 1), pl.ds(i, 1)], lsems[slot])

    def drain_extract(slot):
        pltpu.make_async_copy(
            pslabs[slot].at[:, pl.ds(0, 1)],
            shp.at[tid, :, pl.ds(0, 1)], lsems[slot]).wait()
        pltpu.make_async_copy(
            qslabs[slot].at[:, pl.ds(0, 1)],
            shq.at[tid, :, pl.ds(0, 1)], lsems[slot]).wait()
        pltpu.make_async_copy(
            bslabs[slot].at[:, pl.ds(0, 1)],
            shb.at[tid, pl.ds(0, 1), pl.ds(0, 1)], lsems[slot]).wait()
        pltpu.make_async_copy(
            cslabs[slot].at[:, pl.ds(0, 1)],
            shb.at[tid, pl.ds(1, 1), pl.ds(0, 1)], lsems[slot]).wait()

    def group(g, carry):
        goff = pl.multiple_of(g * 16, 16)
        uv = uidx_v[pl.ds(goff, 16)]
        vv = iidx_v[pl.ds(goff, 16)]
        for j in range(NBUF):
            fire(uv[j], vv[j], j)
        for j in range(16):
            slot = j % NBUF
            wait_slabs(slot)
            extract(uv[j], vv[j], goff + j, slot)
            drain_extract(slot)
            if j + NBUF < 16:
                fire(uv[j + NBUF], vv[j + NBUF], slot)
        return carry

    lax.fori_loop(0, _BPW // 16, group, 0)

    pltpu.make_async_copy(zbuf, shb.at[tid, pl.ds(2, BTAIL - 2), :],
                          zsem).wait()
    pltpu.sync_copy(shp.at[tid], pta_out.at[:, pl.ds(base, _BPW)])
    pltpu.sync_copy(shq.at[tid], qta_out.at[:, pl.ds(base, _BPW)])
    pltpu.sync_copy(shb.at[tid], tail_out.at[:, pl.ds(base, _BPW)])


_gather = pl.kernel(
    _gather_body,
    out_type=(
        jax.ShapeDtypeStruct((EMBED, BATCH), jnp.float32),
        jax.ShapeDtypeStruct((EMBED, BATCH), jnp.float32),
        jax.ShapeDtypeStruct((BTAIL, BATCH), jnp.float32),
    ),
    mesh=plsc.VectorSubcoreMesh(core_axis_name="c", subcore_axis_name="s"),
    scratch_types=[
        pltpu.VMEM((_BPW,), jnp.int32),
        pltpu.VMEM((_BPW,), jnp.int32),
        pltpu.VMEM((BTAIL - 2, LANE), jnp.float32),
        [pltpu.VMEM((EMBED, LANE), jnp.float32) for _ in range(NBUF)],
        [pltpu.VMEM((EMBED, LANE), jnp.float32) for _ in range(NBUF)],
        [pltpu.VMEM((1, LANE), jnp.float32) for _ in range(NBUF)],
        [pltpu.VMEM((1, LANE), jnp.float32) for _ in range(NBUF)],
        pltpu.VMEM_SHARED((_NS, EMBED, LANE), jnp.float32),
        pltpu.VMEM_SHARED((_NS, EMBED, LANE), jnp.float32),
        pltpu.VMEM_SHARED((_NS, BTAIL, LANE), jnp.float32),
        [pltpu.SemaphoreType.DMA for _ in range(NBUF)],
        [pltpu.SemaphoreType.DMA for _ in range(NBUF)],
        [pltpu.SemaphoreType.DMA for _ in range(NBUF)],
        [pltpu.SemaphoreType.DMA for _ in range(NBUF)],
        [pltpu.SemaphoreType.DMA for _ in range(NBUF)],
        pltpu.SemaphoreType.DMA,
    ],
    compiler_params=pltpu.CompilerParams(use_tc_tiling_on_sc=True),
)


_BM = 512


def _score_body(p_ref, q_ref, t_ref, o_ref):
    acc = lax.dot_general(
        p_ref[...], q_ref[...],
        (((0,), (0,)), ((), ())),
        preferred_element_type=jnp.float32,
    )
    ones = jnp.ones((BTAIL, BATCH), jnp.float32)
    o_ref[...] = acc + lax.dot_general(
        t_ref[...], ones,
        (((0,), (0,)), ((), ())),
        preferred_element_type=jnp.float32,
    )


@jax.jit
def _score(pta, qta, tail):
    return pl.pallas_call(
        _score_body,
        grid=(BATCH // _BM,),
        in_specs=[
            pl.BlockSpec((EMBED, _BM), lambda i: (0, i)),
            pl.BlockSpec((EMBED, BATCH), lambda i: (0, 0)),
            pl.BlockSpec((BTAIL, _BM), lambda i: (0, i)),
        ],
        out_specs=pl.BlockSpec((_BM, BATCH), lambda i: (i, 0)),
        out_shape=jax.ShapeDtypeStruct((BATCH, BATCH), jnp.float32),
        compiler_params=pltpu.CompilerParams(
            dimension_semantics=("arbitrary",),
        ),
    )(pta, qta, tail)


@jax.jit
def kernel(user_ids, item_ids, P, Q, B_user, B_item):
    uid = user_ids.astype(jnp.int32)
    iid = item_ids.astype(jnp.int32)
    pta, qta, tail = _gather(P.T, Q.T, B_user.T, B_item.T, uid, iid)
    return _score(pta, qta, tail)


# speedup vs baseline: 1.0451x; 1.0281x over previous
import functools

import jax
import jax.numpy as jnp
from jax import lax
from jax.experimental import pallas as pl
from jax.experimental.pallas import tpu as pltpu
from jax.experimental.pallas import tpu_sc as plsc

BATCH = 4096
EMBED = 32
LANE = 128
NBUF = 4
BTAIL = 8
M_USERS = 1000000
N_ITEMS = 100000

_info = plsc.get_sparse_core_info()
_NC = _info.num_cores
_NS = _info.num_subcores
_NW = _NC * _NS
_BPW = BATCH // _NW


def _gather_body(pt_hbm, qt_hbm, but_hbm, bit_hbm, uid_hbm, iid_hbm,
                 pta_out, qta_out, tail_out,
                 uidx_v, iidx_v, bbuf,
                 pslabs, qslabs, shp, shq, sbu, sbi,
                 psems, qsems, lsems, bsem):
    tid = lax.axis_index("s")
    wid = tid * _NC + lax.axis_index("c")
    base = pl.multiple_of(wid * _BPW, _BPW)

    @pl.when(tid == 0)
    def _():
        pltpu.sync_copy(but_hbm, sbu)

    @pl.when(tid == 1)
    def _():
        pltpu.sync_copy(bit_hbm, sbi)

    pltpu.sync_copy(uid_hbm.at[pl.ds(base, _BPW)], uidx_v)
    pltpu.sync_copy(iid_hbm.at[pl.ds(base, _BPW)], iidx_v)

    zeros = jnp.zeros((16,), jnp.float32)
    for r in range(2, BTAIL):
        for j in range(LANE // 16):
            bbuf[r, pl.ds(j * 16, 16)] = zeros

    plsc.subcore_barrier()

    def fire(u, v, slot):
        uoff = pl.multiple_of((u >> 7) * LANE, LANE)
        voff = pl.multiple_of((v >> 7) * LANE, LANE)
        pltpu.async_copy(pt_hbm.at[:, pl.ds(uoff, LANE)], pslabs[slot],
                         psems[slot])
        pltpu.async_copy(qt_hbm.at[:, pl.ds(voff, LANE)], qslabs[slot],
                         qsems[slot])

    def wait_slabs(slot):
        pltpu.make_async_copy(pt_hbm.at[:, pl.ds(0, LANE)],
                              pslabs[slot], psems[slot]).wait()
        pltpu.make_async_copy(qt_hbm.at[:, pl.ds(0, LANE)],
                              qslabs[slot], qsems[slot]).wait()

    def extract(u, v, i, slot):
        ul = u & (LANE - 1)
        vl = v & (LANE - 1)
        pltpu.async_copy(pslabs[slot].at[:, pl.ds(ul, 1)],
                         shp.at[tid, :, pl.ds(i, 1)], lsems[slot])
        pltpu.async_copy(qslabs[slot].at[:, pl.ds(vl, 1)],
                         shq.at[tid, :, pl.ds(i, 1)], lsems[slot])
        pltpu.async_copy(sbu.at[:, pl.ds(u, 1)],
                         bbuf.at[pl.ds(0, 1), pl.ds(i, 1)], bsem)
        pltpu.async_copy(sbi.at[:, pl.ds(v, 1)],
                         bbuf.at[pl.ds(1, 1), pl.ds(i, 1)], bsem)

    def drain_extract(slot):
        pltpu.make_async_copy(
            pslabs[slot].at[:, pl.ds(0, 1)],
            shp.at[tid, :, pl.ds(0, 1)], lsems[slot]).wait()
        pltpu.make_async_copy(
            qslabs[slot].at[:, pl.ds(0, 1)],
            shq.at[tid, :, pl.ds(0, 1)], lsems[slot]).wait()

    def group(g, carry):
        goff = pl.multiple_of(g * 16, 16)
        uv = uidx_v[pl.ds(goff, 16)]
        vv = iidx_v[pl.ds(goff, 16)]
        for j in range(NBUF):
            fire(uv[j], vv[j], j)
        for j in range(16):
            slot = j % NBUF
            wait_slabs(slot)
            extract(uv[j], vv[j], goff + j, slot)
            drain_extract(slot)
            if j + NBUF < 16:
                fire(uv[j + NBUF], vv[j + NBUF], slot)
        return carry

    lax.fori_loop(0, _BPW // 16, group, 0)

    for _ in range(2 * (_BPW // 16)):
        pltpu.make_async_copy(
            sbu.at[:, pl.ds(0, 16)],
            bbuf.at[pl.ds(0, 1), pl.ds(0, 16)], bsem).wait()

    pltpu.sync_copy(shp.at[tid], pta_out.at[:, pl.ds(base, _BPW)])
    pltpu.sync_copy(shq.at[tid], qta_out.at[:, pl.ds(base, _BPW)])
    pltpu.sync_copy(bbuf, tail_out.at[:, pl.ds(base, _BPW)])


_gather = pl.kernel(
    _gather_body,
    out_type=(
        jax.ShapeDtypeStruct((EMBED, BATCH), jnp.float32),
        jax.ShapeDtypeStruct((EMBED, BATCH), jnp.float32),
        jax.ShapeDtypeStruct((BTAIL, BATCH), jnp.float32),
    ),
    mesh=plsc.VectorSubcoreMesh(core_axis_name="c", subcore_axis_name="s"),
    scratch_types=[
        pltpu.VMEM((_BPW,), jnp.int32),
        pltpu.VMEM((_BPW,), jnp.int32),
        pltpu.VMEM((BTAIL, LANE), jnp.float32),
        [pltpu.VMEM((EMBED, LANE), jnp.float32) for _ in range(NBUF)],
        [pltpu.VMEM((EMBED, LANE), jnp.float32) for _ in range(NBUF)],
        pltpu.VMEM_SHARED((_NS, EMBED, LANE), jnp.float32),
        pltpu.VMEM_SHARED((_NS, EMBED, LANE), jnp.float32),
        pltpu.VMEM_SHARED((1, M_USERS), jnp.float32),
        pltpu.VMEM_SHARED((1, N_ITEMS), jnp.float32),
        [pltpu.SemaphoreType.DMA for _ in range(NBUF)],
        [pltpu.SemaphoreType.DMA for _ in range(NBUF)],
        [pltpu.SemaphoreType.DMA for _ in range(NBUF)],
        pltpu.SemaphoreType.DMA,
    ],
    compiler_params=pltpu.CompilerParams(use_tc_tiling_on_sc=True),
)


_BM = 512


def _score_body(p_ref, q_ref, t_ref, o_ref):
    acc = lax.dot_general(
        p_ref[...], q_ref[...],
        (((0,), (0,)), ((), ())),
        preferred_element_type=jnp.float32,
    )
    ones = jnp.ones((BTAIL, BATCH), jnp.float32)
    o_ref[...] = acc + lax.dot_general(
        t_ref[...], ones,
        (((0,), (0,)), ((), ())),
        preferred_element_type=jnp.float32,
    )


@jax.jit
def _score(pta, qta, tail):
    return pl.pallas_call(
        _score_body,
        grid=(BATCH // _BM,),
        in_specs=[
            pl.BlockSpec((EMBED, _BM), lambda i: (0, i)),
            pl.BlockSpec((EMBED, BATCH), lambda i: (0, 0)),
            pl.BlockSpec((BTAIL, _BM), lambda i: (0, i)),
        ],
        out_specs=pl.BlockSpec((_BM, BATCH), lambda i: (i, 0)),
        out_shape=jax.ShapeDtypeStruct((BATCH, BATCH), jnp.float32),
        compiler_params=pltpu.CompilerParams(
            dimension_semantics=("arbitrary",),
        ),
    )(pta, qta, tail)


@jax.jit
def kernel(user_ids, item_ids, P, Q, B_user, B_item):
    uid = user_ids.astype(jnp.int32)
    iid = item_ids.astype(jnp.int32)
    pta, qta, tail = _gather(P.T, Q.T, B_user.T, B_item.T, uid, iid)
    return _score(pta, qta, tail)
